# baseline (speedup 1.0000x reference)
import functools

import jax
import jax.numpy as jnp
from jax import lax
from jax.experimental import pallas as pl
from jax.experimental.pallas import tpu as pltpu
from jax.experimental.pallas import tpu_sc as plsc

B = 16384
D = 16
NC = 2
NS = 16
NW = NC * NS
BPW = B // NW
CHUNK = 128
NCHUNK = BPW // CHUNK


def _sc_gather_dot(uemb, memb, ubias, mbias, uidx, midx):
    mesh = plsc.VectorSubcoreMesh(core_axis_name="c", subcore_axis_name="s")

    @functools.partial(
        pl.kernel,
        mesh=mesh,
        compiler_params=pltpu.CompilerParams(use_tc_tiling_on_sc=False),
        out_type=[
            jax.ShapeDtypeStruct((NW, D), jnp.float32),
            jax.ShapeDtypeStruct((B,), jnp.float32),
            jax.ShapeDtypeStruct((B,), jnp.float32),
        ],
        scratch_types=[
            pltpu.VMEM((NCHUNK, CHUNK), jnp.int32),
            pltpu.VMEM((NCHUNK, CHUNK), jnp.int32),
            pltpu.VMEM((BPW, D), jnp.float32),
            pltpu.VMEM((BPW, D), jnp.float32),
            pltpu.VMEM((BPW,), jnp.float32),
            pltpu.VMEM((BPW,), jnp.float32),
            pltpu.VMEM((D,), jnp.float32),
            pltpu.SemaphoreType.DMA,
        ],
    )
    def k(uemb_h, memb_h, ubias_h, mbias_h, uidx_h, midx_h,
          part_o, ub_o, mb_o,
          uidx_v, midx_v, urows_v, mrows_v, ubv, mbv, accv, sem):
        wid = lax.axis_index("s") * NC + lax.axis_index("c")
        base = wid * BPW
        pltpu.sync_copy(uidx_h.at[wid], uidx_v)
        pltpu.sync_copy(midx_h.at[wid], midx_v)
        copies = []
        for j in range(NCHUNK):
            sl = pl.ds(j * CHUNK, CHUNK)
            copies.append(pltpu.async_copy(uemb_h.at[uidx_v.at[j]], urows_v.at[sl], sem))
            copies.append(pltpu.async_copy(memb_h.at[midx_v.at[j]], mrows_v.at[sl], sem))
            copies.append(pltpu.async_copy(ubias_h.at[uidx_v.at[j]], ubv.at[sl], sem))
            copies.append(pltpu.async_copy(mbias_h.at[midx_v.at[j]], mbv.at[sl], sem))
        for c in copies:
            c.wait()

        def body(i, acc):
            return acc + urows_v[i] * mrows_v[i]

        acc = lax.fori_loop(0, BPW, body, jnp.zeros((D,), jnp.float32))
        accv[...] = acc
        pltpu.sync_copy(accv, part_o.at[wid])
        pltpu.sync_copy(ubv, ub_o.at[pl.ds(base, BPW)])
        pltpu.sync_copy(mbv, mb_o.at[pl.ds(base, BPW)])

    return k(uemb, memb, ubias, mbias, uidx, midx)


def _tc_epilogue(partials, ub, mb):
    def body(p_ref, ub_ref, mb_ref, o_ref):
        s = jnp.sum(p_ref[...])
        x = s + ub_ref[...] + mb_ref[...]
        o_ref[...] = 1.0 / (1.0 + jnp.exp(-x))

    return pl.pallas_call(
        body,
        out_shape=jax.ShapeDtypeStruct((B // 128, 128), jnp.float32),
    )(partials, ub.reshape(B // 128, 128), mb.reshape(B // 128, 128))


def kernel(inputs, user_embedding, user_bias, movie_embedding, movie_bias):
    idx = inputs.astype(jnp.int32)
    uidx = idx[:, 0].reshape(NW, NCHUNK, CHUNK)
    midx = idx[:, 1].reshape(NW, NCHUNK, CHUNK)
    partials, ub, mb = _sc_gather_dot(
        user_embedding, movie_embedding,
        user_bias.reshape(-1), movie_bias.reshape(-1),
        uidx, midx,
    )
    out = _tc_epilogue(partials, ub, mb)
    return out.reshape(B, 1)


# speedup vs baseline: 3.7685x; 3.7685x over previous
import functools

import jax
import jax.numpy as jnp
from jax import lax
from jax.experimental import pallas as pl
from jax.experimental.pallas import tpu as pltpu
from jax.experimental.pallas import tpu_sc as plsc

B = 16384
D = 16
NC = 2
NS = 16
NW = NC * NS
BPW = B // NW
CHUNK = 128
NCHUNK = BPW // CHUNK
W = 8
NWAVES = BPW // W
LANES = 128


def _sc_gather_dot(uembT, membT, ubias, mbias, uidx3, midx3):
    mesh = plsc.VectorSubcoreMesh(core_axis_name="c", subcore_axis_name="s")

    @functools.partial(
        pl.kernel,
        mesh=mesh,
        compiler_params=pltpu.CompilerParams(
            use_tc_tiling_on_sc=True, needs_layout_passes=False),
        out_type=[
            jax.ShapeDtypeStruct((NW, D), jnp.float32),
            jax.ShapeDtypeStruct((B,), jnp.float32),
            jax.ShapeDtypeStruct((B,), jnp.float32),
        ],
        scratch_types=[
            pltpu.VMEM((NCHUNK, CHUNK), jnp.int32),
            pltpu.VMEM((NCHUNK, CHUNK), jnp.int32),
            pltpu.VMEM((2 * W, D, LANES), jnp.float32),
            pltpu.VMEM((2 * W, D, LANES), jnp.float32),
            pltpu.VMEM((BPW,), jnp.float32),
            pltpu.VMEM((BPW,), jnp.float32),
            pltpu.VMEM((D,), jnp.float32),
            pltpu.SemaphoreType.DMA,
            pltpu.SemaphoreType.DMA,
            pltpu.SemaphoreType.DMA,
            pltpu.SemaphoreType.DMA,
            pltpu.SemaphoreType.DMA,
        ],
    )
    def k(uembT_h, membT_h, ubias_h, mbias_h, uidx3_h, midx3_h,
          part_o, ub_o, mb_o,
          uidx_v, midx_v, slab_u, slab_m, ubv, mbv, accv,
          sem_u0, sem_u1, sem_m0, sem_m1, sem_b):
        wid = lax.axis_index("s") * NC + lax.axis_index("c")
        base = wid * BPW
        pltpu.sync_copy(uidx3_h.at[wid], uidx_v)
        pltpu.sync_copy(midx3_h.at[wid], midx_v)

        bias_copies = []
        for j in range(NCHUNK):
            sl = pl.ds(j * CHUNK, CHUNK)
            bias_copies.append(
                pltpu.async_copy(ubias_h.at[uidx_v.at[j]], ubv.at[sl], sem_b))
            bias_copies.append(
                pltpu.async_copy(mbias_h.at[midx_v.at[j]], mbv.at[sl], sem_b))

        dvec = lax.iota(jnp.int32, D)

        def idxvec(ref, pair):
            return ref[pair >> 3, pl.ds((pair & 7) * 16, 16)]

        def fire(uv, mv, lane0, slot, su, sm):
            for i in range(W):
                ru = uv[lane0 + i]
                rm = mv[lane0 + i]
                offu = pl.multiple_of((ru >> 7) * LANES, LANES)
                offm = pl.multiple_of((rm >> 7) * LANES, LANES)
                pltpu.async_copy(
                    uembT_h.at[:, pl.ds(offu, LANES)],
                    slab_u.at[slot * W + i], su)
                pltpu.async_copy(
                    membT_h.at[:, pl.ds(offm, LANES)],
                    slab_m.at[slot * W + i], sm)

        def drain(slot, su, sm):
            for i in range(W):
                pltpu.make_async_copy(
                    uembT_h.at[:, pl.ds(0, LANES)],
                    slab_u.at[slot * W + i], su).wait()
                pltpu.make_async_copy(
                    membT_h.at[:, pl.ds(0, LANES)],
                    slab_m.at[slot * W + i], sm).wait()

        def extract(uv, mv, lane0, slot, acc):
            for i in range(W):
                ru = uv[lane0 + i]
                rm = mv[lane0 + i]
                lu = jnp.full((D,), ru & 127, jnp.int32)
                lm = jnp.full((D,), rm & 127, jnp.int32)
                u = plsc.load_gather(slab_u.at[slot * W + i], [dvec, lu])
                m = plsc.load_gather(slab_m.at[slot * W + i], [dvec, lm])
                acc = acc + u * m
            return acc

        uv0 = idxvec(uidx_v, 0)
        mv0 = idxvec(midx_v, 0)
        fire(uv0, mv0, 0, 0, sem_u0, sem_m0)
        fire(uv0, mv0, W, 1, sem_u1, sem_m1)

        NPAIR = NWAVES // 2

        def body(p, acc):
            uvp = idxvec(uidx_v, p)
            mvp = idxvec(midx_v, p)
            drain(0, sem_u0, sem_m0)
            acc = extract(uvp, mvp, 0, 0, acc)

            @pl.when(p < NPAIR - 1)
            def _():
                uvn = idxvec(uidx_v, p + 1)
                mvn = idxvec(midx_v, p + 1)
                fire(uvn, mvn, 0, 0, sem_u0, sem_m0)

            drain(1, sem_u1, sem_m1)
            acc = extract(uvp, mvp, W, 1, acc)

            @pl.when(p < NPAIR - 1)
            def _():
                uvn = idxvec(uidx_v, p + 1)
                mvn = idxvec(midx_v, p + 1)
                fire(uvn, mvn, W, 1, sem_u1, sem_m1)

            return acc

        acc = lax.fori_loop(0, NPAIR, body,
                            jnp.zeros((D,), jnp.float32))
        accv[...] = acc
        pltpu.sync_copy(accv, part_o.at[wid])
        for c in bias_copies:
            c.wait()
        pltpu.sync_copy(ubv, ub_o.at[pl.ds(base, BPW)])
        pltpu.sync_copy(mbv, mb_o.at[pl.ds(base, BPW)])

    return k(uembT, membT, ubias, mbias, uidx3, midx3)


def _tc_epilogue(partials, ub, mb):
    def body(p_ref, ub_ref, mb_ref, o_ref):
        s = jnp.sum(p_ref[...])
        x = s + ub_ref[...] + mb_ref[...]
        o_ref[...] = 1.0 / (1.0 + jnp.exp(-x))

    return pl.pallas_call(
        body,
        out_shape=jax.ShapeDtypeStruct((B // 128, 128), jnp.float32),
    )(partials, ub.reshape(B // 128, 128), mb.reshape(B // 128, 128))


def kernel(inputs, user_embedding, user_bias, movie_embedding, movie_bias):
    idx = inputs.astype(jnp.int32)
    uidx = idx[:, 0]
    midx = idx[:, 1]
    partials, ub, mb = _sc_gather_dot(
        user_embedding.T, movie_embedding.T,
        user_bias.reshape(-1), movie_bias.reshape(-1),
        uidx.reshape(NW, NCHUNK, CHUNK), midx.reshape(NW, NCHUNK, CHUNK),
    )
    out = _tc_epilogue(partials, ub, mb)
    return out.reshape(B, 1)


# speedup vs baseline: 6.0063x; 1.5938x over previous
import functools

import jax
import jax.numpy as jnp
from jax import lax
from jax.experimental import pallas as pl
from jax.experimental.pallas import tpu as pltpu
from jax.experimental.pallas import tpu_sc as plsc

B = 16384
D = 16
NC = 2
NS = 16
NW = NC * NS
BPW = B // NW
CHUNK = 128
NCHUNK = BPW // CHUNK
W = 8
NPAIR = BPW // 16
LANES = 128


def _sc_gather_dot(uembT, membT, ubiasT, mbiasT, uidx3, midx3):
    mesh = plsc.VectorSubcoreMesh(core_axis_name="c", subcore_axis_name="s")

    @functools.partial(
        pl.kernel,
        mesh=mesh,
        compiler_params=pltpu.CompilerParams(
            use_tc_tiling_on_sc=True, needs_layout_passes=False),
        out_type=[
            jax.ShapeDtypeStruct((NW, D), jnp.float32),
            jax.ShapeDtypeStruct((B,), jnp.float32),
            jax.ShapeDtypeStruct((B,), jnp.float32),
        ],
        scratch_types=[
            pltpu.VMEM((NCHUNK, CHUNK), jnp.int32),
            pltpu.VMEM((NCHUNK, CHUNK), jnp.int32),
            pltpu.VMEM((2 * W, D, LANES), jnp.float32),
            pltpu.VMEM((2 * W, D, LANES), jnp.float32),
            pltpu.VMEM((2, 16, LANES), jnp.float32),
            pltpu.VMEM((2, 16, LANES), jnp.float32),
            pltpu.VMEM((BPW,), jnp.float32),
            pltpu.VMEM((BPW,), jnp.float32),
            pltpu.VMEM((D,), jnp.float32),
            pltpu.SemaphoreType.DMA,
            pltpu.SemaphoreType.DMA,
            pltpu.SemaphoreType.DMA,
            pltpu.SemaphoreType.DMA,
            pltpu.SemaphoreType.DMA,
            pltpu.SemaphoreType.DMA,
        ],
    )
    def k(uembT_h, membT_h, ubiasT_h, mbiasT_h, uidx3_h, midx3_h,
          part_o, ub_o, mb_o,
          uidx_v, midx_v, slab_u, slab_m, bslab_u, bslab_m, ubv, mbv, accv,
          sem_u0, sem_u1, sem_m0, sem_m1, sem_b0, sem_b1):
        wid = lax.axis_index("s") * NC + lax.axis_index("c")
        base = wid * BPW
        pltpu.sync_copy(uidx3_h.at[wid], uidx_v)
        pltpu.sync_copy(midx3_h.at[wid], midx_v)

        dvec = lax.iota(jnp.int32, D)

        def idxvec(ref, pair):
            return ref[pair >> 3, pl.ds((pair & 7) * 16, 16)]

        def fire(uv, mv, lane0, slot, su, sm):
            for i in range(W):
                ru = uv[lane0 + i]
                rm = mv[lane0 + i]
                offu = pl.multiple_of((ru >> 7) * LANES, LANES)
                offm = pl.multiple_of((rm >> 7) * LANES, LANES)
                pltpu.async_copy(
                    uembT_h.at[:, pl.ds(offu, LANES)],
                    slab_u.at[slot * W + i], su)
                pltpu.async_copy(
                    membT_h.at[:, pl.ds(offm, LANES)],
                    slab_m.at[slot * W + i], sm)

        def fire_bias(uv, mv, par, sb):
            for i in range(16):
                ru = uv[i]
                rm = mv[i]
                offu = pl.multiple_of((ru >> 7) * LANES, LANES)
                offm = pl.multiple_of((rm >> 7) * LANES, LANES)
                pltpu.async_copy(
                    ubiasT_h.at[:, pl.ds(offu, LANES)],
                    bslab_u.at[par].at[pl.ds(i, 1)], sb)
                pltpu.async_copy(
                    mbiasT_h.at[:, pl.ds(offm, LANES)],
                    bslab_m.at[par].at[pl.ds(i, 1)], sb)

        def drain(slot, su, sm):
            for i in range(W):
                pltpu.make_async_copy(
                    uembT_h.at[:, pl.ds(0, LANES)],
                    slab_u.at[slot * W + i], su).wait()
                pltpu.make_async_copy(
                    membT_h.at[:, pl.ds(0, LANES)],
                    slab_m.at[slot * W + i], sm).wait()

        def drain_bias(par, sb):
            for i in range(16):
                pltpu.make_async_copy(
                    ubiasT_h.at[:, pl.ds(0, LANES)],
                    bslab_u.at[par].at[pl.ds(i, 1)], sb).wait()
                pltpu.make_async_copy(
                    mbiasT_h.at[:, pl.ds(0, LANES)],
                    bslab_m.at[par].at[pl.ds(i, 1)], sb).wait()

        def extract(uv, mv, lane0, slot, acc):
            for i in range(W):
                ru = uv[lane0 + i]
                rm = mv[lane0 + i]
                lu = jnp.full((D,), ru & 127, jnp.int32)
                lm = jnp.full((D,), rm & 127, jnp.int32)
                u = plsc.load_gather(slab_u.at[slot * W + i], [dvec, lu])
                m = plsc.load_gather(slab_m.at[slot * W + i], [dvec, lm])
                acc = acc + u * m
            return acc

        def extract_bias(uv, mv, pair, par):
            ub16 = plsc.load_gather(bslab_u.at[par], [dvec, uv & 127])
            mb16 = plsc.load_gather(bslab_m.at[par], [dvec, mv & 127])
            off = pl.multiple_of(pair * 16, 16)
            ubv[pl.ds(off, 16)] = ub16
            mbv[pl.ds(off, 16)] = mb16

        uv0 = idxvec(uidx_v, 0)
        mv0 = idxvec(midx_v, 0)
        uv1 = idxvec(uidx_v, 1)
        mv1 = idxvec(midx_v, 1)
        fire(uv0, mv0, 0, 0, sem_u0, sem_m0)
        fire(uv0, mv0, W, 1, sem_u1, sem_m1)
        fire_bias(uv0, mv0, 0, sem_b0)
        fire_bias(uv1, mv1, 1, sem_b1)

        def pair_step(p, acc, sb, par):
            uvp = idxvec(uidx_v, p)
            mvp = idxvec(midx_v, p)
            drain(0, sem_u0, sem_m0)
            acc = extract(uvp, mvp, 0, 0, acc)

            @pl.when(p < NPAIR - 1)
            def _():
                uvn = idxvec(uidx_v, p + 1)
                mvn = idxvec(midx_v, p + 1)
                fire(uvn, mvn, 0, 0, sem_u0, sem_m0)

            drain(1, sem_u1, sem_m1)
            acc = extract(uvp, mvp, W, 1, acc)

            @pl.when(p < NPAIR - 1)
            def _():
                uvn = idxvec(uidx_v, p + 1)
                mvn = idxvec(midx_v, p + 1)
                fire(uvn, mvn, W, 1, sem_u1, sem_m1)

            drain_bias(par, sb)
            extract_bias(uvp, mvp, p, par)

            @pl.when(p < NPAIR - 2)
            def _():
                uvn = idxvec(uidx_v, p + 2)
                mvn = idxvec(midx_v, p + 2)
                fire_bias(uvn, mvn, par, sb)

            return acc

        def body(q, acc):
            acc = pair_step(2 * q, acc, sem_b0, 0)
            acc = pair_step(2 * q + 1, acc, sem_b1, 1)
            return acc

        acc = lax.fori_loop(0, NPAIR // 2, body,
                            jnp.zeros((D,), jnp.float32))
        accv[...] = acc
        pltpu.sync_copy(accv, part_o.at[wid])
        pltpu.sync_copy(ubv, ub_o.at[pl.ds(base, BPW)])
        pltpu.sync_copy(mbv, mb_o.at[pl.ds(base, BPW)])

    return k(uembT, membT, ubiasT, mbiasT, uidx3, midx3)


def _tc_epilogue(partials, ub, mb):
    def body(p_ref, ub_ref, mb_ref, o_ref):
        s = jnp.sum(p_ref[...])
        x = s + ub_ref[...] + mb_ref[...]
        o_ref[...] = 1.0 / (1.0 + jnp.exp(-x))

    return pl.pallas_call(
        body,
        out_shape=jax.ShapeDtypeStruct((B // 128, 128), jnp.float32),
    )(partials, ub.reshape(B // 128, 128), mb.reshape(B // 128, 128))


def kernel(inputs, user_embedding, user_bias, movie_embedding, movie_bias):
    idx = inputs.astype(jnp.int32)
    uidx = idx[:, 0]
    midx = idx[:, 1]
    partials, ub, mb = _sc_gather_dot(
        user_embedding.T, movie_embedding.T,
        user_bias.T, movie_bias.T,
        uidx.reshape(NW, NCHUNK, CHUNK), midx.reshape(NW, NCHUNK, CHUNK),
    )
    out = _tc_epilogue(partials, ub, mb)
    return out.reshape(B, 1)


# speedup vs baseline: 6.2145x; 1.0347x over previous
import functools

import jax
import jax.numpy as jnp
from jax import lax
from jax.experimental import pallas as pl
from jax.experimental.pallas import tpu as pltpu
from jax.experimental.pallas import tpu_sc as plsc

B = 16384
D = 16
NC = 2
NS = 16
NW = NC * NS
BPW = B // NW
CHUNK = 128
NCHUNK = BPW // CHUNK
W = 8
NPAIR = BPW // 16
LANES = 128


def _sc_gather_dot(uembT, membT, ubiasT, mbiasT, uidx3, midx3):
    mesh = plsc.VectorSubcoreMesh(core_axis_name="c", subcore_axis_name="s")

    @functools.partial(
        pl.kernel,
        mesh=mesh,
        compiler_params=pltpu.CompilerParams(
            use_tc_tiling_on_sc=True, needs_layout_passes=False),
        out_type=[
            jax.ShapeDtypeStruct((NW, D), jnp.float32),
            jax.ShapeDtypeStruct((B,), jnp.float32),
            jax.ShapeDtypeStruct((B,), jnp.float32),
        ],
        scratch_types=[
            pltpu.VMEM((NCHUNK, CHUNK), jnp.int32),
            pltpu.VMEM((NCHUNK, CHUNK), jnp.int32),
            pltpu.VMEM((3 * W, D, LANES), jnp.float32),
            pltpu.VMEM((3 * W, D, LANES), jnp.float32),
            pltpu.VMEM((2, 16, LANES), jnp.float32),
            pltpu.VMEM((2, 16, LANES), jnp.float32),
            pltpu.VMEM((BPW,), jnp.float32),
            pltpu.VMEM((BPW,), jnp.float32),
            pltpu.VMEM((D,), jnp.float32),
            pltpu.SemaphoreType.DMA,
            pltpu.SemaphoreType.DMA,
            pltpu.SemaphoreType.DMA,
            pltpu.SemaphoreType.DMA,
            pltpu.SemaphoreType.DMA,
            pltpu.SemaphoreType.DMA,
            pltpu.SemaphoreType.DMA,
            pltpu.SemaphoreType.DMA,
        ],
    )
    def k(uembT_h, membT_h, ubiasT_h, mbiasT_h, uidx3_h, midx3_h,
          part_o, ub_o, mb_o,
          uidx_v, midx_v, slab_u, slab_m, bslab_u, bslab_m, ubv, mbv, accv,
          sem_u0, sem_u1, sem_u2, sem_m0, sem_m1, sem_m2, sem_b0, sem_b1):
        wid = lax.axis_index("s") * NC + lax.axis_index("c")
        base = wid * BPW
        pltpu.sync_copy(uidx3_h.at[wid], uidx_v)
        pltpu.sync_copy(midx3_h.at[wid], midx_v)

        dvec = lax.iota(jnp.int32, D)

        def idxvec(ref, pair):
            return ref[pair >> 3, pl.ds((pair & 7) * 16, 16)]

        def fire(uv, mv, lane0, slot, su, sm):
            for i in range(W):
                ru = uv[lane0 + i]
                rm = mv[lane0 + i]
                offu = pl.multiple_of((ru >> 7) * LANES, LANES)
                offm = pl.multiple_of((rm >> 7) * LANES, LANES)
                pltpu.async_copy(
                    uembT_h.at[:, pl.ds(offu, LANES)],
                    slab_u.at[slot * W + i], su)
                pltpu.async_copy(
                    membT_h.at[:, pl.ds(offm, LANES)],
                    slab_m.at[slot * W + i], sm)

        def fire_bias(uv, mv, par, sb):
            for i in range(16):
                ru = uv[i]
                rm = mv[i]
                offu = pl.multiple_of((ru >> 7) * LANES, LANES)
                offm = pl.multiple_of((rm >> 7) * LANES, LANES)
                pltpu.async_copy(
                    ubiasT_h.at[:, pl.ds(offu, LANES)],
                    bslab_u.at[par].at[pl.ds(i, 1)], sb)
                pltpu.async_copy(
                    mbiasT_h.at[:, pl.ds(offm, LANES)],
                    bslab_m.at[par].at[pl.ds(i, 1)], sb)

        def drain(slot, su, sm):
            for i in range(W):
                pltpu.make_async_copy(
                    uembT_h.at[:, pl.ds(0, LANES)],
                    slab_u.at[slot * W + i], su).wait()
                pltpu.make_async_copy(
                    membT_h.at[:, pl.ds(0, LANES)],
                    slab_m.at[slot * W + i], sm).wait()

        def drain_bias(par, sb):
            for i in range(16):
                pltpu.make_async_copy(
                    ubiasT_h.at[:, pl.ds(0, LANES)],
                    bslab_u.at[par].at[pl.ds(i, 1)], sb).wait()
                pltpu.make_async_copy(
                    mbiasT_h.at[:, pl.ds(0, LANES)],
                    bslab_m.at[par].at[pl.ds(i, 1)], sb).wait()

        def extract(uv, mv, lane0, slot, acc):
            for i in range(W):
                ru = uv[lane0 + i]
                rm = mv[lane0 + i]
                lu = jnp.full((D,), ru & 127, jnp.int32)
                lm = jnp.full((D,), rm & 127, jnp.int32)
                u = plsc.load_gather(slab_u.at[slot * W + i], [dvec, lu])
                m = plsc.load_gather(slab_m.at[slot * W + i], [dvec, lm])
                acc = acc + u * m
            return acc

        def extract_bias(uv, mv, pair, par):
            ub16 = plsc.load_gather(bslab_u.at[par], [dvec, uv & 127])
            mb16 = plsc.load_gather(bslab_m.at[par], [dvec, mv & 127])
            off = pl.multiple_of(pair * 16, 16)
            ubv[pl.ds(off, 16)] = ub16
            mbv[pl.ds(off, 16)] = mb16

        sems_u = (sem_u0, sem_u1, sem_u2)
        sems_m = (sem_m0, sem_m1, sem_m2)
        sems_b = (sem_b0, sem_b1)
        NWAVE = 2 * NPAIR

        acc0 = jnp.zeros((D,), jnp.float32)
        uvp0 = idxvec(uidx_v, 0)
        mvp0 = idxvec(midx_v, 0)
        uvp1 = idxvec(uidx_v, 1)
        mvp1 = idxvec(midx_v, 1)
        fire(uvp0, mvp0, 0, 0, sem_u0, sem_m0)
        fire(uvp0, mvp0, W, 1, sem_u1, sem_m1)
        fire(uvp1, mvp1, 0, 2, sem_u2, sem_m2)
        fire_bias(uvp0, mvp0, 0, sem_b0)
        fire_bias(uvp1, mvp1, 1, sem_b1)

        def step(w, slot, lane0, acc):
            uvp = idxvec(uidx_v, w >> 1)
            mvp = idxvec(midx_v, w >> 1)
            drain(slot, sems_u[slot], sems_m[slot])
            acc = extract(uvp, mvp, lane0, slot, acc)

            @pl.when(w + 3 < NWAVE)
            def _():
                wn = w + 3
                uvn = idxvec(uidx_v, wn >> 1)
                mvn = idxvec(midx_v, wn >> 1)
                if lane0 == 0:
                    fire(uvn, mvn, W, slot, sems_u[slot], sems_m[slot])
                else:
                    fire(uvn, mvn, 0, slot, sems_u[slot], sems_m[slot])

            return acc

        def bias_step(p, par, acc_unused=None):
            uvp = idxvec(uidx_v, p)
            mvp = idxvec(midx_v, p)
            drain_bias(par, sems_b[par])
            extract_bias(uvp, mvp, p, par)

            @pl.when(p + 2 < NPAIR)
            def _():
                uvn = idxvec(uidx_v, p + 2)
                mvn = idxvec(midx_v, p + 2)
                fire_bias(uvn, mvn, par, sems_b[par])

        def body(q, acc):
            w0 = 12 * q
            for j in range(12):
                acc = step(w0 + j, j % 3, (j % 2) * W, acc)
                if j % 2 == 1:
                    bias_step((w0 + j) >> 1, ((j // 2) % 2))
            return acc

        acc = lax.fori_loop(0, 5, body, acc0)
        for j in range(4):
            w = 60 + j
            acc = step(w, (60 + j) % 3, (j % 2) * W, acc)
            if j % 2 == 1:
                bias_step(w >> 1, (30 + j // 2) % 2)
        accv[...] = acc
        pltpu.sync_copy(accv, part_o.at[wid])
        pltpu.sync_copy(ubv, ub_o.at[pl.ds(base, BPW)])
        pltpu.sync_copy(mbv, mb_o.at[pl.ds(base, BPW)])

    return k(uembT, membT, ubiasT, mbiasT, uidx3, midx3)


def _tc_epilogue(partials, ub, mb):
    def body(p_ref, ub_ref, mb_ref, o_ref):
        s = jnp.sum(p_ref[...])
        x = s + ub_ref[...] + mb_ref[...]
        o_ref[...] = 1.0 / (1.0 + jnp.exp(-x))

    return pl.pallas_call(
        body,
        out_shape=jax.ShapeDtypeStruct((B // 128, 128), jnp.float32),
    )(partials, ub.reshape(B // 128, 128), mb.reshape(B // 128, 128))


def kernel(inputs, user_embedding, user_bias, movie_embedding, movie_bias):
    idx = inputs.astype(jnp.int32)
    uidx = idx[:, 0]
    midx = idx[:, 1]
    partials, ub, mb = _sc_gather_dot(
        user_embedding.T, movie_embedding.T,
        user_bias.T, movie_bias.T,
        uidx.reshape(NW, NCHUNK, CHUNK), midx.reshape(NW, NCHUNK, CHUNK),
    )
    out = _tc_epilogue(partials, ub, mb)
    return out.reshape(B, 1)


# speedup vs baseline: 6.2628x; 1.0078x over previous
import functools

import jax
import jax.numpy as jnp
from jax import lax
from jax.experimental import pallas as pl
from jax.experimental.pallas import tpu as pltpu
from jax.experimental.pallas import tpu_sc as plsc

B = 16384
D = 16
NC = 2
NS = 16
NW = NC * NS
BPW = B // NW
CHUNK = 128
NCHUNK = BPW // CHUNK
W = 8
NPAIR = BPW // 16
LANES = 128


def _sc_gather_dot(uembT, membT, ubiasT, mbiasT, uidx3, midx3):
    mesh = plsc.VectorSubcoreMesh(core_axis_name="c", subcore_axis_name="s")

    @functools.partial(
        pl.kernel,
        mesh=mesh,
        compiler_params=pltpu.CompilerParams(
            use_tc_tiling_on_sc=True, needs_layout_passes=False),
        out_type=[
            jax.ShapeDtypeStruct((NW, D), jnp.float32),
            jax.ShapeDtypeStruct((B,), jnp.float32),
            jax.ShapeDtypeStruct((B,), jnp.float32),
        ],
        scratch_types=[
            pltpu.VMEM((NCHUNK, CHUNK), jnp.int32),
            pltpu.VMEM((NCHUNK, CHUNK), jnp.int32),
            pltpu.VMEM((3 * W, D, LANES), jnp.float32),
            pltpu.VMEM((3 * W, D, LANES), jnp.float32),
            pltpu.VMEM((2, 16, LANES), jnp.float32),
            pltpu.VMEM((2, 16, LANES), jnp.float32),
            pltpu.VMEM((BPW,), jnp.float32),
            pltpu.VMEM((BPW,), jnp.float32),
            pltpu.VMEM((D,), jnp.float32),
            pltpu.SemaphoreType.DMA,
            pltpu.SemaphoreType.DMA,
            pltpu.SemaphoreType.DMA,
            pltpu.SemaphoreType.DMA,
            pltpu.SemaphoreType.DMA,
            pltpu.SemaphoreType.DMA,
            pltpu.SemaphoreType.DMA,
            pltpu.SemaphoreType.DMA,
        ],
    )
    def k(uembT_h, membT_h, ubiasT_h, mbiasT_h, uidx3_h, midx3_h,
          part_o, ub_o, mb_o,
          uidx_v, midx_v, slab_u, slab_m, bslab_u, bslab_m, ubv, mbv, accv,
          sem_u0, sem_u1, sem_u2, sem_m0, sem_m1, sem_m2, sem_b0, sem_b1):
        wid = lax.axis_index("s") * NC + lax.axis_index("c")
        base = wid * BPW
        pltpu.sync_copy(uidx3_h.at[wid], uidx_v)
        pltpu.sync_copy(midx3_h.at[wid], midx_v)

        dvec = lax.iota(jnp.int32, D)

        def idxvec(ref, pair):
            return ref[pair >> 3, pl.ds((pair & 7) * 16, 16)]

        def fire(uv, mv, lane0, slot, su, sm):
            for i in range(W):
                ru = uv[lane0 + i]
                rm = mv[lane0 + i]
                offu = pl.multiple_of((ru >> 7) * LANES, LANES)
                offm = pl.multiple_of((rm >> 7) * LANES, LANES)
                pltpu.async_copy(
                    uembT_h.at[:, pl.ds(offu, LANES)],
                    slab_u.at[slot * W + i], su)
                pltpu.async_copy(
                    membT_h.at[:, pl.ds(offm, LANES)],
                    slab_m.at[slot * W + i], sm)

        def fire_bias(uv, mv, par, sb):
            for i in range(16):
                ru = uv[i]
                rm = mv[i]
                offu = pl.multiple_of((ru >> 7) * LANES, LANES)
                offm = pl.multiple_of((rm >> 7) * LANES, LANES)
                pltpu.async_copy(
                    ubiasT_h.at[:, pl.ds(offu, LANES)],
                    bslab_u.at[par].at[pl.ds(i, 1)], sb)
                pltpu.async_copy(
                    mbiasT_h.at[:, pl.ds(offm, LANES)],
                    bslab_m.at[par].at[pl.ds(i, 1)], sb)

        def drain(slot, su, sm):
            for i in range(W):
                pltpu.make_async_copy(
                    uembT_h.at[:, pl.ds(0, LANES)],
                    slab_u.at[slot * W + i], su).wait()
                pltpu.make_async_copy(
                    membT_h.at[:, pl.ds(0, LANES)],
                    slab_m.at[slot * W + i], sm).wait()

        def drain_bias(par, sb):
            for i in range(16):
                pltpu.make_async_copy(
                    ubiasT_h.at[:, pl.ds(0, LANES)],
                    bslab_u.at[par].at[pl.ds(i, 1)], sb).wait()
                pltpu.make_async_copy(
                    mbiasT_h.at[:, pl.ds(0, LANES)],
                    bslab_m.at[par].at[pl.ds(i, 1)], sb).wait()

        def extract(uv, mv, lane0, slot, acc):
            for i in range(W):
                ru = uv[lane0 + i]
                rm = mv[lane0 + i]
                lu = jnp.full((D,), ru & 127, jnp.int32)
                lm = jnp.full((D,), rm & 127, jnp.int32)
                u = plsc.load_gather(slab_u.at[slot * W + i], [dvec, lu])
                m = plsc.load_gather(slab_m.at[slot * W + i], [dvec, lm])
                acc = acc + u * m
            return acc

        def extract_bias(uv, mv, pair, par):
            ub16 = plsc.load_gather(bslab_u.at[par], [dvec, uv & 127])
            mb16 = plsc.load_gather(bslab_m.at[par], [dvec, mv & 127])
            off = pl.multiple_of(pair * 16, 16)
            ubv[pl.ds(off, 16)] = ub16
            mbv[pl.ds(off, 16)] = mb16

        sems_u = (sem_u0, sem_u1, sem_u2)
        sems_m = (sem_m0, sem_m1, sem_m2)
        sems_b = (sem_b0, sem_b1)
        NWAVE = 2 * NPAIR

        acc0 = jnp.zeros((D,), jnp.float32)
        uvp0 = idxvec(uidx_v, 0)
        mvp0 = idxvec(midx_v, 0)
        uvp1 = idxvec(uidx_v, 1)
        mvp1 = idxvec(midx_v, 1)
        fire(uvp0, mvp0, 0, 0, sem_u0, sem_m0)
        fire(uvp0, mvp0, W, 1, sem_u1, sem_m1)
        fire(uvp1, mvp1, 0, 2, sem_u2, sem_m2)
        fire_bias(uvp0, mvp0, 0, sem_b0)
        fire_bias(uvp1, mvp1, 1, sem_b1)

        def step(w, slot, lane0, acc):
            uvp = idxvec(uidx_v, w >> 1)
            mvp = idxvec(midx_v, w >> 1)
            drain(slot, sems_u[slot], sems_m[slot])
            acc = extract(uvp, mvp, lane0, slot, acc)

            @pl.when(w + 3 < NWAVE)
            def _():
                wn = w + 3
                uvn = idxvec(uidx_v, wn >> 1)
                mvn = idxvec(midx_v, wn >> 1)
                if lane0 == 0:
                    fire(uvn, mvn, W, slot, sems_u[slot], sems_m[slot])
                else:
                    fire(uvn, mvn, 0, slot, sems_u[slot], sems_m[slot])

            return acc

        def bias_step(p, par):
            uvp = idxvec(uidx_v, p)
            mvp = idxvec(midx_v, p)
            drain_bias(par, sems_b[par])
            extract_bias(uvp, mvp, p, par)

            @pl.when(p + 2 < NPAIR)
            def _():
                uvn = idxvec(uidx_v, p + 2)
                mvn = idxvec(midx_v, p + 2)
                fire_bias(uvn, mvn, par, sems_b[par])

        def body(q, acc):
            w0 = 12 * q
            for j in range(12):
                acc = step(w0 + j, j % 3, (j % 2) * W, acc)
                if j % 2 == 1:
                    bias_step((w0 + j) >> 1, ((j // 2) % 2))
            return acc

        acc = lax.fori_loop(0, 5, body, acc0)
        for j in range(4):
            w = 60 + j
            acc = step(w, (60 + j) % 3, (j % 2) * W, acc)
            if j % 2 == 1:
                bias_step(w >> 1, (30 + j // 2) % 2)
        accv[...] = acc
        pltpu.sync_copy(accv, part_o.at[wid])
        pltpu.sync_copy(ubv, ub_o.at[pl.ds(base, BPW)])
        pltpu.sync_copy(mbv, mb_o.at[pl.ds(base, BPW)])

    return k(uembT, membT, ubiasT, mbiasT, uidx3, midx3)


def _tc_epilogue(partials, ub, mb):
    def body(p_ref, ub_ref, mb_ref, o_ref):
        s = jnp.sum(p_ref[...])
        x = s + ub_ref[...] + mb_ref[...]
        o_ref[...] = 1.0 / (1.0 + jnp.exp(-x))

    return pl.pallas_call(
        body,
        out_shape=jax.ShapeDtypeStruct((B // 128, 128), jnp.float32),
    )(partials, ub.reshape(B // 128, 128), mb.reshape(B // 128, 128))


def kernel(inputs, user_embedding, user_bias, movie_embedding, movie_bias):
    idx = inputs.astype(jnp.int32)
    uidx = idx[:, 0]
    midx = idx[:, 1]
    partials, ub, mb = _sc_gather_dot(
        user_embedding.T, movie_embedding.T,
        user_bias.T, movie_bias.T,
        uidx.reshape(NW, NCHUNK, CHUNK), midx.reshape(NW, NCHUNK, CHUNK),
    )
    out = _tc_epilogue(partials, ub, mb)
    return out.reshape(B, 1)
